# baseline (speedup 1.0000x reference)
import functools

import jax
import jax.numpy as jnp
from jax import lax
from jax.experimental import pallas as pl
from jax.experimental.pallas import tpu as pltpu
from jax.experimental.pallas import tpu_sc as plsc

_N = 10000
_E = 320000
_D = 128
_R = 4

_NC = 2
_NS = 16
_NW = _NC * _NS
_EPW = _E // _NW
_C = 128
_EPWP = 10240
_NCH = _EPWP // _C
_PAD_TYP = 3
_PAD_DST = _N
_RN = _R * _N
_RNP = 40960
_NPT = 640
_DH = _D // 2
_NB = 4


def _mm_body(x_ref, w_ref, b_ref, oa_ref, ob_ref):
    res = (jnp.dot(x_ref[...], w_ref[0],
                   preferred_element_type=jnp.float32) + b_ref[0, 0])
    oa_ref[0] = res[:, :_DH]
    ob_ref[0] = res[:, _DH:]


def _mm(x, wcat, bcat):
    bn = 1000
    half = jax.ShapeDtypeStruct((_R + 1, _N, _DH), jnp.float32)
    return pl.pallas_call(
        _mm_body,
        grid=(_R + 1, _N // bn),
        in_specs=[pl.BlockSpec((bn, _D), lambda r, j: (j, 0)),
                  pl.BlockSpec((1, _D, _D), lambda r, j: (r, 0, 0)),
                  pl.BlockSpec((1, 1, _D), lambda r, j: (r, 0, 0))],
        out_specs=(pl.BlockSpec((1, bn, _DH), lambda r, j: (r, j, 0)),
                   pl.BlockSpec((1, bn, _DH), lambda r, j: (r, j, 0))),
        out_shape=(half, half),
    )(x, wcat, bcat)


def _mid_body(ra_ref, rb_ref, aa_ref, ab_ref, w_ref, b_ref, oa_ref, ob_ref):
    ha = jnp.maximum(ra_ref[0] + aa_ref[0, 0] + aa_ref[0, 1], 0.0)
    hb = jnp.maximum(rb_ref[0] + ab_ref[0, 0] + ab_ref[0, 1], 0.0)
    w = w_ref[0]
    res = (jnp.dot(ha, w[:_DH, :], preferred_element_type=jnp.float32)
           + jnp.dot(hb, w[_DH:, :], preferred_element_type=jnp.float32)
           + b_ref[0, 0])
    oa_ref[0] = res[:, :_DH]
    ob_ref[0] = res[:, _DH:]


def _mid(ha, hb, agg, wcat, bcat):
    bn = 1000
    halfspec = pl.BlockSpec((1, bn, _DH), lambda r, j: (_R, j, 0))
    half = jax.ShapeDtypeStruct((_R + 1, _N, _DH), jnp.float32)
    return pl.pallas_call(
        _mid_body,
        grid=(_R + 1, _N // bn),
        in_specs=[halfspec, halfspec,
                  pl.BlockSpec((1, 2, bn, _DH), lambda r, j: (0, 0, j, 0)),
                  pl.BlockSpec((1, 2, bn, _DH), lambda r, j: (1, 0, j, 0)),
                  pl.BlockSpec((1, _D, _D), lambda r, j: (r, 0, 0)),
                  pl.BlockSpec((1, 1, _D), lambda r, j: (r, 0, 0))],
        out_specs=(pl.BlockSpec((1, bn, _DH), lambda r, j: (r, j, 0)),
                   pl.BlockSpec((1, bn, _DH), lambda r, j: (r, j, 0))),
        out_shape=(half, half),
    )(ha, hb, agg, agg, wcat, bcat)


def _inv_body(cnt_ref, inv_ref):
    s = cnt_ref[0] + cnt_ref[1]
    inv_ref[...] = 1.0 / jnp.maximum(s, 1.0)


def _inv(cnt3):
    return pl.pallas_call(
        _inv_body,
        out_shape=jax.ShapeDtypeStruct(cnt3.shape[1:], jnp.float32),
    )(cnt3)


def _fin_body(ra_ref, rb_ref, aa_ref, ab_ref, w_ref, b_ref, o_ref):
    ha = jnp.maximum(ra_ref[0] + aa_ref[0, 0] + aa_ref[0, 1], 0.0)
    hb = jnp.maximum(rb_ref[0] + ab_ref[0, 0] + ab_ref[0, 1], 0.0)
    w = w_ref[...]
    o_ref[...] = (jnp.dot(ha, w[:_DH, :], preferred_element_type=jnp.float32)
                  + jnp.dot(hb, w[_DH:, :], preferred_element_type=jnp.float32)
                  + b_ref[0])


def _fin(ha, hb, agg, wc_pad, bc_pad):
    bn = 1000
    halfspec = pl.BlockSpec((1, bn, _DH), lambda j: (_R, j, 0))
    return pl.pallas_call(
        _fin_body,
        grid=(_N // bn,),
        in_specs=[halfspec, halfspec,
                  pl.BlockSpec((1, 2, bn, _DH), lambda j: (0, 0, j, 0)),
                  pl.BlockSpec((1, 2, bn, _DH), lambda j: (1, 0, j, 0)),
                  pl.BlockSpec((_D, _D), lambda j: (0, 0)),
                  pl.BlockSpec((1, _D), lambda j: (0, 0))],
        out_specs=pl.BlockSpec((bn, _D), lambda j: (j, 0)),
        out_shape=jax.ShapeDtypeStruct((_N, _D), jnp.float32),
    )(ha, hb, agg, agg, wc_pad, bc_pad)


def _sc_mesh():
    return plsc.VectorSubcoreMesh(core_axis_name="c", subcore_axis_name="s")


_SC_PARAMS = pltpu.CompilerParams(needs_layout_passes=False,
                                  use_tc_tiling_on_sc=False)


def _counts(dstp, typp, zeros_rnp):
    @functools.partial(
        pl.kernel,
        mesh=_sc_mesh(),
        compiler_params=_SC_PARAMS,
        out_type=jax.ShapeDtypeStruct((_NC, _RNP), jnp.float32),
        scratch_types=[
            pltpu.VMEM((_C,), jnp.int32),
            pltpu.VMEM((_C,), jnp.int32),
            pltpu.VMEM((_C,), jnp.int32),
            pltpu.VMEM((_C,), jnp.float32),
            pltpu.VMEM_SHARED((_RNP,), jnp.float32),
        ],
    )
    def k(dst_h, typ_h, z_h, out_h, d_v, t_v, idx_v, ones_v, cnt_sh):
        c = lax.axis_index("c")
        s = lax.axis_index("s")
        wid = c * _NS + s
        ebase = wid * _EPWP
        ones16 = jnp.full((16,), 1.0, dtype=jnp.float32)
        for i in range(_C // 16):
            ones_v[pl.ds(i * 16, 16)] = ones16

        @pl.when(s == 0)
        def _():
            pltpu.sync_copy(z_h, cnt_sh)

        plsc.subcore_barrier()

        def chunk2(i, carry):
            base = ebase + i * _C
            pltpu.sync_copy(dst_h.at[pl.ds(base, _C)], d_v)
            pltpu.sync_copy(typ_h.at[pl.ds(base, _C)], t_v)
            for k2 in range(_C // 16):
                sl = pl.ds(k2 * 16, 16)
                idx_v[sl] = t_v[sl] * _N + d_v[sl]
            pltpu.sync_copy(ones_v, cnt_sh.at[idx_v], add=True)
            return carry

        lax.fori_loop(0, _NCH, chunk2, 0)
        plsc.subcore_barrier()

        @pl.when(s == 0)
        def _():
            pltpu.sync_copy(cnt_sh, out_h.at[c])

    return k(dstp, typp, zeros_rnp)


@functools.lru_cache(maxsize=1)
def _agg_kernel():
    @functools.partial(
        pl.kernel,
        mesh=_sc_mesh(),
        compiler_params=_SC_PARAMS,
        out_type=jax.ShapeDtypeStruct((2, _NC, _N, _DH), jnp.float32),
        scratch_types=[
            pltpu.VMEM((_NB, _C), jnp.int32),
            pltpu.VMEM((_NB, _C), jnp.int32),
            pltpu.VMEM((_NB, _C), jnp.int32),
            pltpu.VMEM((_NB, _C), jnp.int32),
            pltpu.VMEM((_NB, _C), jnp.int32),
            pltpu.VMEM((_NB, _C, _DH), jnp.float32),
            pltpu.VMEM((_NB, _C, 16), jnp.float32),
            pltpu.SemaphoreType.DMA((_NB,)),
            pltpu.SemaphoreType.DMA((_NB,)),
            pltpu.SemaphoreType.DMA((_NB,)),
            pltpu.SemaphoreType.DMA((_NB,)),
            pltpu.VMEM_SHARED((_N + 16, _DH), jnp.float32),
        ],
    )
    def k(hra_h, hrb_h, invb_h, src_h, typ_h, dst_h, z_h, out_h,
          srcr, typr, sidx, cidx, didx, rows_v, srow_v,
          sem_i, sem_g, sem_c, sem_s, acc_sh):
        c = lax.axis_index("c")
        s = lax.axis_index("s")
        wid = c * _NS + s
        ebase = wid * _EPWP
        rb = jnp.minimum(s * _NPT, _N - _NPT)

        for half, hr_h in ((0, hra_h), (1, hrb_h)):

            def fill_and_gather(i, b, hr_h=hr_h):
                base = ebase + i * _C
                a1 = pltpu.async_copy(src_h.at[pl.ds(base, _C)],
                                      srcr.at[b], sem_i.at[b])
                a2 = pltpu.async_copy(typ_h.at[pl.ds(base, _C)],
                                      typr.at[b], sem_i.at[b])
                a3 = pltpu.async_copy(dst_h.at[pl.ds(base, _C)],
                                      didx.at[b], sem_i.at[b])
                a1.wait()
                a2.wait()
                a3.wait()
                for k2 in range(_C // 16):
                    sl = pl.ds(k2 * 16, 16)
                    t16 = typr[b, sl] * _N
                    sidx[b, sl] = t16 + srcr[b, sl]
                    cidx[b, sl] = t16 + didx[b, sl]
                pltpu.async_copy(hr_h.at[sidx.at[b]], rows_v.at[b],
                                 sem_g.at[b])
                pltpu.async_copy(invb_h.at[cidx.at[b]], srow_v.at[b],
                                 sem_c.at[b])

            def wait_gathers(b, hr_h=hr_h):
                pltpu.make_async_copy(hr_h.at[sidx.at[b]], rows_v.at[b],
                                      sem_g.at[b]).wait()
                pltpu.make_async_copy(invb_h.at[cidx.at[b]], srow_v.at[b],
                                      sem_c.at[b]).wait()

            def wait_scatter(b):
                pltpu.make_async_copy(rows_v.at[b],
                                      acc_sh.at[didx.at[b]],
                                      sem_s.at[b]).wait()

            pltpu.sync_copy(z_h.at[pl.ds(rb, _NPT)],
                            acc_sh.at[pl.ds(rb, _NPT)])
            plsc.subcore_barrier()
            fill_and_gather(0, 0)
            fill_and_gather(1, 1)

            def chunk(i, carry):
                b = lax.bitwise_and(i, _NB - 1)
                nb = lax.bitwise_and(i + 2, _NB - 1)

                @pl.when(i + 2 < _NCH)
                def _():
                    @pl.when(i >= 2)
                    def _():
                        wait_scatter(nb)
                    fill_and_gather(i + 2, nb)

                wait_gathers(b)

                def rowgrp(g, c2):
                    for jj in range(4):
                        row = g * 4 + jj
                        srow = srow_v[b, row, :]
                        for k3 in range(_DH // 16):
                            sl = pl.ds(k3 * 16, 16)
                            rows_v[b, row, sl] = rows_v[b, row, sl] * srow
                    return c2

                lax.fori_loop(0, _C // 4, rowgrp, 0)
                pltpu.async_copy(rows_v.at[b], acc_sh.at[didx.at[b]],
                                 sem_s.at[b], add=True)
                return carry

            lax.fori_loop(0, _NCH, chunk, 0)
            for b in range(_NB):
                wait_scatter(b)
            plsc.subcore_barrier()
            pltpu.sync_copy(acc_sh.at[pl.ds(rb, _NPT)],
                            out_h.at[half, c, pl.ds(rb, _NPT)])
            plsc.subcore_barrier()

    return k


def _agg(hra, hrb, invb, srcp, typp, dstp, zeros_ndh):
    return _agg_kernel()(hra, hrb, invb, srcp, typp, dstp, zeros_ndh)


def kernel(x, edge_index, edge_type, W1_rel, W1_root, b1,
           W2_rel, W2_root, b2, Wc, bc):
    npad = _EPWP - _EPW
    srcp = jnp.concatenate(
        [edge_index[0].reshape(_NW, _EPW),
         jnp.zeros((_NW, npad), jnp.int32)], axis=1).reshape(-1)
    dstp = jnp.concatenate(
        [edge_index[1].reshape(_NW, _EPW),
         jnp.full((_NW, npad), _PAD_DST, jnp.int32)], axis=1).reshape(-1)
    typp = jnp.concatenate(
        [edge_type.reshape(_NW, _EPW),
         jnp.full((_NW, npad), _PAD_TYP, jnp.int32)], axis=1).reshape(-1)

    zeros_rnp = jnp.zeros((_RNP,), jnp.float32)
    zeros_ndh = jnp.zeros((_N, _DH), jnp.float32)

    cnt = _counts(dstp, typp, zeros_rnp)
    inv = _inv(cnt.reshape(_NC, _RNP // _D, _D)).reshape(_RNP)
    invb = jnp.broadcast_to(inv[:, None], (_RNP, 16))

    wcat1 = jnp.concatenate([W1_rel, W1_root[None]], axis=0)
    bcat1 = jnp.zeros((_R + 1, 1, _D), jnp.float32).at[_R, 0].set(b1)
    ha1, hb1 = _mm(x, wcat1, bcat1)
    hra1 = ha1[:_R].reshape(_RN, _DH)
    hrb1 = hb1[:_R].reshape(_RN, _DH)
    agg1 = _agg(hra1, hrb1, invb, srcp, typp, dstp, zeros_ndh)

    wcat2 = jnp.concatenate([W2_rel, W2_root[None]], axis=0)
    bcat2 = jnp.zeros((_R + 1, 1, _D), jnp.float32).at[_R, 0].set(b2)
    ha2, hb2 = _mid(ha1, hb1, agg1, wcat2, bcat2)
    hra2 = ha2[:_R].reshape(_RN, _DH)
    hrb2 = hb2[:_R].reshape(_RN, _DH)
    agg2 = _agg(hra2, hrb2, invb, srcp, typp, dstp, zeros_ndh)

    wc_pad = jnp.pad(Wc, ((0, 0), (0, _D - Wc.shape[1])))
    bc_pad = jnp.pad(bc, (0, _D - bc.shape[0])).reshape(1, _D)
    out = _fin(ha2, hb2, agg2, wc_pad, bc_pad)
    return out[:, :Wc.shape[1]]


# speedup vs baseline: 1.5346x; 1.5346x over previous
import functools

import jax
import jax.numpy as jnp
from jax import lax
from jax.experimental import pallas as pl
from jax.experimental.pallas import tpu as pltpu
from jax.experimental.pallas import tpu_sc as plsc

_N = 10000
_E = 320000
_D = 128
_R = 4

_NC = 2
_NS = 16
_NW = _NC * _NS
_EPW = _E // _NW
_C = 80
_NCH = _EPW // _C
_RN = _R * _N
_RNP = 40960
_NPT = 640
_DH = _D // 2
_NB = 4


def _gidx_body(src_ref, dst_ref, typ_ref, gsrc_ref, gdst_ref):
    t = typ_ref[...] * _N
    gsrc_ref[...] = t + src_ref[...]
    gdst_ref[...] = t + dst_ref[...]


def _gidx(src2, dst2, typ2):
    return pl.pallas_call(
        _gidx_body,
        out_shape=(jax.ShapeDtypeStruct(src2.shape, jnp.int32),
                   jax.ShapeDtypeStruct(src2.shape, jnp.int32)),
    )(src2, dst2, typ2)


def _mm_body(x_ref, w_ref, b_ref, oa_ref, ob_ref):
    res = (jnp.dot(x_ref[...], w_ref[0],
                   preferred_element_type=jnp.float32) + b_ref[0, 0])
    oa_ref[0] = res[:, :_DH]
    ob_ref[0] = res[:, _DH:]


def _mm(x, wcat, bcat):
    bn = 1000
    half = jax.ShapeDtypeStruct((_R + 1, _N, _DH), jnp.float32)
    return pl.pallas_call(
        _mm_body,
        grid=(_R + 1, _N // bn),
        in_specs=[pl.BlockSpec((bn, _D), lambda r, j: (j, 0)),
                  pl.BlockSpec((1, _D, _D), lambda r, j: (r, 0, 0)),
                  pl.BlockSpec((1, 1, _D), lambda r, j: (r, 0, 0))],
        out_specs=(pl.BlockSpec((1, bn, _DH), lambda r, j: (r, j, 0)),
                   pl.BlockSpec((1, bn, _DH), lambda r, j: (r, j, 0))),
        out_shape=(half, half),
    )(x, wcat, bcat)


def _mid_body(ra_ref, rb_ref, aa_ref, ab_ref, w_ref, b_ref, oa_ref, ob_ref):
    ha = jnp.maximum(ra_ref[0] + aa_ref[0, 0] + aa_ref[0, 1], 0.0)
    hb = jnp.maximum(rb_ref[0] + ab_ref[0, 0] + ab_ref[0, 1], 0.0)
    w = w_ref[0]
    res = (jnp.dot(ha, w[:_DH, :], preferred_element_type=jnp.float32)
           + jnp.dot(hb, w[_DH:, :], preferred_element_type=jnp.float32)
           + b_ref[0, 0])
    oa_ref[0] = res[:, :_DH]
    ob_ref[0] = res[:, _DH:]


def _mid(ha, hb, agg, wcat, bcat):
    bn = 1000
    halfspec = pl.BlockSpec((1, bn, _DH), lambda r, j: (_R, j, 0))
    half = jax.ShapeDtypeStruct((_R + 1, _N, _DH), jnp.float32)
    return pl.pallas_call(
        _mid_body,
        grid=(_R + 1, _N // bn),
        in_specs=[halfspec, halfspec,
                  pl.BlockSpec((1, 2, bn, _DH), lambda r, j: (0, 0, j, 0)),
                  pl.BlockSpec((1, 2, bn, _DH), lambda r, j: (1, 0, j, 0)),
                  pl.BlockSpec((1, _D, _D), lambda r, j: (r, 0, 0)),
                  pl.BlockSpec((1, 1, _D), lambda r, j: (r, 0, 0))],
        out_specs=(pl.BlockSpec((1, bn, _DH), lambda r, j: (r, j, 0)),
                   pl.BlockSpec((1, bn, _DH), lambda r, j: (r, j, 0))),
        out_shape=(half, half),
    )(ha, hb, agg, agg, wcat, bcat)


def _inv_body(cnt_ref, inv_ref):
    s = cnt_ref[0] + cnt_ref[1]
    inv_ref[...] = 1.0 / jnp.maximum(s, 1.0)


def _inv(cnt3):
    return pl.pallas_call(
        _inv_body,
        out_shape=jax.ShapeDtypeStruct(cnt3.shape[1:], jnp.float32),
    )(cnt3)


def _fin_body(ra_ref, rb_ref, aa_ref, ab_ref, w_ref, b_ref, o_ref):
    ha = jnp.maximum(ra_ref[0] + aa_ref[0, 0] + aa_ref[0, 1], 0.0)
    hb = jnp.maximum(rb_ref[0] + ab_ref[0, 0] + ab_ref[0, 1], 0.0)
    w = w_ref[...]
    o_ref[...] = (jnp.dot(ha, w[:_DH, :], preferred_element_type=jnp.float32)
                  + jnp.dot(hb, w[_DH:, :], preferred_element_type=jnp.float32)
                  + b_ref[0])


def _fin(ha, hb, agg, wc_pad, bc_pad):
    bn = 1000
    halfspec = pl.BlockSpec((1, bn, _DH), lambda j: (_R, j, 0))
    return pl.pallas_call(
        _fin_body,
        grid=(_N // bn,),
        in_specs=[halfspec, halfspec,
                  pl.BlockSpec((1, 2, bn, _DH), lambda j: (0, 0, j, 0)),
                  pl.BlockSpec((1, 2, bn, _DH), lambda j: (1, 0, j, 0)),
                  pl.BlockSpec((_D, _D), lambda j: (0, 0)),
                  pl.BlockSpec((1, _D), lambda j: (0, 0))],
        out_specs=pl.BlockSpec((bn, _D), lambda j: (j, 0)),
        out_shape=jax.ShapeDtypeStruct((_N, _D), jnp.float32),
    )(ha, hb, agg, agg, wc_pad, bc_pad)


def _sc_mesh():
    return plsc.VectorSubcoreMesh(core_axis_name="c", subcore_axis_name="s")


_SC_PARAMS = pltpu.CompilerParams(needs_layout_passes=False,
                                  use_tc_tiling_on_sc=False)


def _counts(gdst, zeros_rnp):
    @functools.partial(
        pl.kernel,
        mesh=_sc_mesh(),
        compiler_params=_SC_PARAMS,
        out_type=jax.ShapeDtypeStruct((_NC, _RNP), jnp.float32),
        scratch_types=[
            pltpu.VMEM((_EPW,), jnp.int32),
            pltpu.VMEM((_C,), jnp.int32),
            pltpu.VMEM((_C,), jnp.float32),
            pltpu.VMEM_SHARED((_RNP,), jnp.float32),
        ],
    )
    def k(gdst_h, z_h, out_h, gidx_all, idx_v, ones_v, cnt_sh):
        c = lax.axis_index("c")
        s = lax.axis_index("s")
        wid = c * _NS + s
        pltpu.sync_copy(gdst_h.at[pl.ds(wid * _EPW, _EPW)], gidx_all)
        ones16 = jnp.full((16,), 1.0, dtype=jnp.float32)
        for i in range(_C // 16):
            ones_v[pl.ds(i * 16, 16)] = ones16

        @pl.when(s == 0)
        def _():
            pltpu.sync_copy(z_h, cnt_sh)

        plsc.subcore_barrier()

        def chunk(i, carry):
            cb = i * _C
            for k2 in range(_C // 16):
                idx_v[pl.ds(k2 * 16, 16)] = gidx_all[pl.ds(cb + k2 * 16, 16)]
            pltpu.sync_copy(ones_v, cnt_sh.at[idx_v], add=True)
            return carry

        lax.fori_loop(0, _NCH, chunk, 0)
        plsc.subcore_barrier()

        @pl.when(s == 0)
        def _():
            pltpu.sync_copy(cnt_sh, out_h.at[c])

    return k(gdst, zeros_rnp)


@functools.lru_cache(maxsize=1)
def _agg_kernel():
    @functools.partial(
        pl.kernel,
        mesh=_sc_mesh(),
        compiler_params=_SC_PARAMS,
        out_type=jax.ShapeDtypeStruct((2, _NC, _N, _DH), jnp.float32),
        scratch_types=[
            pltpu.VMEM((_NB, _C), jnp.int32),
            pltpu.VMEM((_NB, _C), jnp.int32),
            pltpu.VMEM((_NB, _C), jnp.int32),
            pltpu.VMEM((_NB, _C, _DH), jnp.float32),
            pltpu.VMEM((_NB, _C, 16), jnp.float32),
            pltpu.SemaphoreType.DMA((_NB,)),
            pltpu.SemaphoreType.DMA((_NB,)),
            pltpu.SemaphoreType.DMA((_NB,)),
            pltpu.SemaphoreType.DMA((_NB,)),
            pltpu.VMEM_SHARED((_N, _DH), jnp.float32),
        ],
    )
    def k(hra_h, hrb_h, invb_h, gsrc_h, gdst_h, dst_h, z_h, out_h,
          sidx, cidx, didx, rows_v, srow_v,
          sem_i, sem_g, sem_c, sem_s, acc_sh):
        c = lax.axis_index("c")
        s = lax.axis_index("s")
        wid = c * _NS + s
        ebase = wid * _EPW
        rb = jnp.minimum(s * _NPT, _N - _NPT)

        for half, hr_h in ((0, hra_h), (1, hrb_h)):

            def fill_and_gather(i, b, hr_h=hr_h):
                base = ebase + i * _C
                a1 = pltpu.async_copy(gsrc_h.at[pl.ds(base, _C)],
                                      sidx.at[b], sem_i.at[b])
                a2 = pltpu.async_copy(gdst_h.at[pl.ds(base, _C)],
                                      cidx.at[b], sem_i.at[b])
                a3 = pltpu.async_copy(dst_h.at[pl.ds(base, _C)],
                                      didx.at[b], sem_i.at[b])
                a1.wait()
                a2.wait()
                a3.wait()
                pltpu.async_copy(hr_h.at[sidx.at[b]], rows_v.at[b],
                                 sem_g.at[b])
                pltpu.async_copy(invb_h.at[cidx.at[b]], srow_v.at[b],
                                 sem_c.at[b])

            def wait_gathers(b, hr_h=hr_h):
                pltpu.make_async_copy(hr_h.at[sidx.at[b]], rows_v.at[b],
                                      sem_g.at[b]).wait()
                pltpu.make_async_copy(invb_h.at[cidx.at[b]], srow_v.at[b],
                                      sem_c.at[b]).wait()

            def wait_scatter(b):
                pltpu.make_async_copy(rows_v.at[b],
                                      acc_sh.at[didx.at[b]],
                                      sem_s.at[b]).wait()

            pltpu.sync_copy(z_h.at[pl.ds(rb, _NPT)],
                            acc_sh.at[pl.ds(rb, _NPT)])
            plsc.subcore_barrier()
            fill_and_gather(0, 0)
            fill_and_gather(1, 1)

            def chunk(i, carry):
                b = lax.bitwise_and(i, _NB - 1)
                nb = lax.bitwise_and(i + 2, _NB - 1)

                @pl.when(i + 2 < _NCH)
                def _():
                    @pl.when(i >= 2)
                    def _():
                        wait_scatter(nb)
                    fill_and_gather(i + 2, nb)

                wait_gathers(b)

                def rowgrp(g, c2):
                    for jj in range(4):
                        row = g * 4 + jj
                        srow = srow_v[b, row, :]
                        for k3 in range(_DH // 16):
                            sl = pl.ds(k3 * 16, 16)
                            rows_v[b, row, sl] = rows_v[b, row, sl] * srow
                    return c2

                lax.fori_loop(0, _C // 4, rowgrp, 0)
                pltpu.async_copy(rows_v.at[b], acc_sh.at[didx.at[b]],
                                 sem_s.at[b], add=True)
                return carry

            lax.fori_loop(0, _NCH, chunk, 0)
            for b in range(_NB):
                wait_scatter(b)
            plsc.subcore_barrier()
            pltpu.sync_copy(acc_sh.at[pl.ds(rb, _NPT)],
                            out_h.at[half, c, pl.ds(rb, _NPT)])
            plsc.subcore_barrier()

    return k


def _agg(hra, hrb, invb, gsrc, gdst, dstv, zeros_ndh):
    return _agg_kernel()(hra, hrb, invb, gsrc, gdst, dstv, zeros_ndh)


def kernel(x, edge_index, edge_type, W1_rel, W1_root, b1,
           W2_rel, W2_root, b2, Wc, bc):
    src = edge_index[0]
    dst = edge_index[1]

    rows = _E // _D
    gsrc2, gdst2 = _gidx(src.reshape(rows, _D), dst.reshape(rows, _D),
                         edge_type.reshape(rows, _D))
    gsrc = gsrc2.reshape(_E)
    gdst = gdst2.reshape(_E)

    zeros_rnp = jnp.zeros((_RNP,), jnp.float32)
    zeros_ndh = jnp.zeros((_N, _DH), jnp.float32)

    cnt = _counts(gdst, zeros_rnp)
    inv = _inv(cnt.reshape(_NC, _RNP // _D, _D)).reshape(_RNP)
    invb = jnp.broadcast_to(inv[:, None], (_RNP, 16))

    wcat1 = jnp.concatenate([W1_rel, W1_root[None]], axis=0)
    bcat1 = jnp.zeros((_R + 1, 1, _D), jnp.float32).at[_R, 0].set(b1)
    ha1, hb1 = _mm(x, wcat1, bcat1)
    hra1 = ha1[:_R].reshape(_RN, _DH)
    hrb1 = hb1[:_R].reshape(_RN, _DH)
    agg1 = _agg(hra1, hrb1, invb, gsrc, gdst, dst, zeros_ndh)

    wcat2 = jnp.concatenate([W2_rel, W2_root[None]], axis=0)
    bcat2 = jnp.zeros((_R + 1, 1, _D), jnp.float32).at[_R, 0].set(b2)
    ha2, hb2 = _mid(ha1, hb1, agg1, wcat2, bcat2)
    hra2 = ha2[:_R].reshape(_RN, _DH)
    hrb2 = hb2[:_R].reshape(_RN, _DH)
    agg2 = _agg(hra2, hrb2, invb, gsrc, gdst, dst, zeros_ndh)

    wc_pad = jnp.pad(Wc, ((0, 0), (0, _D - Wc.shape[1])))
    bc_pad = jnp.pad(bc, (0, _D - bc.shape[0])).reshape(1, _D)
    out = _fin(ha2, hb2, agg2, wc_pad, bc_pad)
    return out[:, :Wc.shape[1]]


# speedup vs baseline: 1.9952x; 1.3001x over previous
import functools

import jax
import jax.numpy as jnp
from jax import lax
from jax.experimental import pallas as pl
from jax.experimental.pallas import tpu as pltpu
from jax.experimental.pallas import tpu_sc as plsc

_N = 10000
_E = 320000
_D = 128
_R = 4

_NC = 2
_NS = 16
_NW = _NC * _NS
_EPW = _E // _NW
_C = 80
_NCH = _EPW // _C
_RN = _R * _N
_RNP = 40960
_NPT = 640
_DH = _D // 2
_NB = 4
_NI = 8


def _gidx_body(src_ref, dst_ref, typ_ref, gsrc_ref, gdst_ref):
    t = typ_ref[...] * _N
    gsrc_ref[...] = t + src_ref[...]
    gdst_ref[...] = t + dst_ref[...]


def _gidx(src2, dst2, typ2):
    return pl.pallas_call(
        _gidx_body,
        out_shape=(jax.ShapeDtypeStruct(src2.shape, jnp.int32),
                   jax.ShapeDtypeStruct(src2.shape, jnp.int32)),
    )(src2, dst2, typ2)


def _mm_body(x_ref, w_ref, b_ref, oa_ref, ob_ref):
    res = (jnp.dot(x_ref[...], w_ref[0],
                   preferred_element_type=jnp.float32) + b_ref[0, 0])
    oa_ref[0] = res[:, :_DH]
    ob_ref[0] = res[:, _DH:]


def _mm(x, wcat, bcat):
    bn = 1000
    half = jax.ShapeDtypeStruct((_R + 1, _N, _DH), jnp.float32)
    return pl.pallas_call(
        _mm_body,
        grid=(_R + 1, _N // bn),
        in_specs=[pl.BlockSpec((bn, _D), lambda r, j: (j, 0)),
                  pl.BlockSpec((1, _D, _D), lambda r, j: (r, 0, 0)),
                  pl.BlockSpec((1, 1, _D), lambda r, j: (r, 0, 0))],
        out_specs=(pl.BlockSpec((1, bn, _DH), lambda r, j: (r, j, 0)),
                   pl.BlockSpec((1, bn, _DH), lambda r, j: (r, j, 0))),
        out_shape=(half, half),
    )(x, wcat, bcat)


def _mid_body(ra_ref, rb_ref, aa_ref, ab_ref, w_ref, b_ref, oa_ref, ob_ref):
    ha = jnp.maximum(ra_ref[0] + aa_ref[0, 0] + aa_ref[0, 1], 0.0)
    hb = jnp.maximum(rb_ref[0] + ab_ref[0, 0] + ab_ref[0, 1], 0.0)
    w = w_ref[0]
    res = (jnp.dot(ha, w[:_DH, :], preferred_element_type=jnp.float32)
           + jnp.dot(hb, w[_DH:, :], preferred_element_type=jnp.float32)
           + b_ref[0, 0])
    oa_ref[0] = res[:, :_DH]
    ob_ref[0] = res[:, _DH:]


def _mid(ha, hb, agg, wcat, bcat):
    bn = 1000
    halfspec = pl.BlockSpec((1, bn, _DH), lambda r, j: (_R, j, 0))
    half = jax.ShapeDtypeStruct((_R + 1, _N, _DH), jnp.float32)
    return pl.pallas_call(
        _mid_body,
        grid=(_R + 1, _N // bn),
        in_specs=[halfspec, halfspec,
                  pl.BlockSpec((1, 2, bn, _DH), lambda r, j: (0, 0, j, 0)),
                  pl.BlockSpec((1, 2, bn, _DH), lambda r, j: (1, 0, j, 0)),
                  pl.BlockSpec((1, _D, _D), lambda r, j: (r, 0, 0)),
                  pl.BlockSpec((1, 1, _D), lambda r, j: (r, 0, 0))],
        out_specs=(pl.BlockSpec((1, bn, _DH), lambda r, j: (r, j, 0)),
                   pl.BlockSpec((1, bn, _DH), lambda r, j: (r, j, 0))),
        out_shape=(half, half),
    )(ha, hb, agg, agg, wcat, bcat)


def _inv_body(cnt_ref, inv_ref):
    s = cnt_ref[0] + cnt_ref[1]
    inv_ref[...] = 1.0 / jnp.maximum(s, 1.0)


def _inv(cnt3):
    return pl.pallas_call(
        _inv_body,
        out_shape=jax.ShapeDtypeStruct(cnt3.shape[1:], jnp.float32),
    )(cnt3)


def _fin_body(ra_ref, rb_ref, aa_ref, ab_ref, w_ref, b_ref, o_ref):
    ha = jnp.maximum(ra_ref[0] + aa_ref[0, 0] + aa_ref[0, 1], 0.0)
    hb = jnp.maximum(rb_ref[0] + ab_ref[0, 0] + ab_ref[0, 1], 0.0)
    w = w_ref[...]
    o_ref[...] = (jnp.dot(ha, w[:_DH, :], preferred_element_type=jnp.float32)
                  + jnp.dot(hb, w[_DH:, :], preferred_element_type=jnp.float32)
                  + b_ref[0])


def _fin(ha, hb, agg, wc_pad, bc_pad):
    bn = 1000
    halfspec = pl.BlockSpec((1, bn, _DH), lambda j: (_R, j, 0))
    return pl.pallas_call(
        _fin_body,
        grid=(_N // bn,),
        in_specs=[halfspec, halfspec,
                  pl.BlockSpec((1, 2, bn, _DH), lambda j: (0, 0, j, 0)),
                  pl.BlockSpec((1, 2, bn, _DH), lambda j: (1, 0, j, 0)),
                  pl.BlockSpec((_D, _D), lambda j: (0, 0)),
                  pl.BlockSpec((1, _D), lambda j: (0, 0))],
        out_specs=pl.BlockSpec((bn, _D), lambda j: (j, 0)),
        out_shape=jax.ShapeDtypeStruct((_N, _D), jnp.float32),
    )(ha, hb, agg, agg, wc_pad, bc_pad)


def _sc_mesh():
    return plsc.VectorSubcoreMesh(core_axis_name="c", subcore_axis_name="s")


_SC_PARAMS = pltpu.CompilerParams(needs_layout_passes=False,
                                  use_tc_tiling_on_sc=False)


def _counts(gdst, zeros_rnp):
    @functools.partial(
        pl.kernel,
        mesh=_sc_mesh(),
        compiler_params=_SC_PARAMS,
        out_type=jax.ShapeDtypeStruct((_NC, _RNP), jnp.float32),
        scratch_types=[
            pltpu.VMEM((_EPW,), jnp.int32),
            pltpu.VMEM((_C,), jnp.int32),
            pltpu.VMEM((_C,), jnp.float32),
            pltpu.VMEM_SHARED((_RNP,), jnp.float32),
        ],
    )
    def k(gdst_h, z_h, out_h, gidx_all, idx_v, ones_v, cnt_sh):
        c = lax.axis_index("c")
        s = lax.axis_index("s")
        wid = c * _NS + s
        pltpu.sync_copy(gdst_h.at[pl.ds(wid * _EPW, _EPW)], gidx_all)
        ones16 = jnp.full((16,), 1.0, dtype=jnp.float32)
        for i in range(_C // 16):
            ones_v[pl.ds(i * 16, 16)] = ones16

        @pl.when(s == 0)
        def _():
            pltpu.sync_copy(z_h, cnt_sh)

        plsc.subcore_barrier()

        def chunk(i, carry):
            cb = i * _C
            for k2 in range(_C // 16):
                idx_v[pl.ds(k2 * 16, 16)] = gidx_all[pl.ds(cb + k2 * 16, 16)]
            pltpu.sync_copy(ones_v, cnt_sh.at[idx_v], add=True)
            return carry

        lax.fori_loop(0, _NCH, chunk, 0)
        plsc.subcore_barrier()

        @pl.when(s == 0)
        def _():
            pltpu.sync_copy(cnt_sh, out_h.at[c])

    return k(gdst, zeros_rnp)


@functools.lru_cache(maxsize=1)
def _agg_kernel():
    @functools.partial(
        pl.kernel,
        mesh=_sc_mesh(),
        compiler_params=_SC_PARAMS,
        out_type=jax.ShapeDtypeStruct((2, _NC, _N, _DH), jnp.float32),
        scratch_types=[
            pltpu.VMEM((_NI, _C), jnp.int32),
            pltpu.VMEM((_NI, _C), jnp.int32),
            pltpu.VMEM((_NI, _C), jnp.int32),
            pltpu.VMEM((_NB, _C, _DH), jnp.float32),
            pltpu.VMEM((_NB, _C, 16), jnp.float32),
            pltpu.SemaphoreType.DMA((_NI,)),
            pltpu.SemaphoreType.DMA((_NB,)),
            pltpu.SemaphoreType.DMA((_NB,)),
            pltpu.SemaphoreType.DMA((_NB,)),
            pltpu.VMEM_SHARED((_N, _DH), jnp.float32),
        ],
    )
    def k(hra_h, hrb_h, invb_h, gsrc_h, gdst_h, dst_h, z_h, out_h,
          sidx, cidx, didx, rows_v, srow_v,
          sem_i, sem_g, sem_c, sem_s, acc_sh):
        c = lax.axis_index("c")
        s = lax.axis_index("s")
        wid = c * _NS + s
        ebase = wid * _EPW
        rb = jnp.minimum(s * _NPT, _N - _NPT)

        for half, hr_h in ((0, hra_h), (1, hrb_h)):

            def idx_dma(j, ib):
                base = ebase + j * _C
                pltpu.async_copy(gsrc_h.at[pl.ds(base, _C)],
                                 sidx.at[ib], sem_i.at[ib])
                pltpu.async_copy(gdst_h.at[pl.ds(base, _C)],
                                 cidx.at[ib], sem_i.at[ib])
                pltpu.async_copy(dst_h.at[pl.ds(base, _C)],
                                 didx.at[ib], sem_i.at[ib])

            def start_gathers(ib, vb, hr_h=hr_h):
                pltpu.make_async_copy(gsrc_h.at[pl.ds(ebase, _C)],
                                      sidx.at[ib], sem_i.at[ib]).wait()
                pltpu.make_async_copy(gdst_h.at[pl.ds(ebase, _C)],
                                      cidx.at[ib], sem_i.at[ib]).wait()
                pltpu.make_async_copy(dst_h.at[pl.ds(ebase, _C)],
                                      didx.at[ib], sem_i.at[ib]).wait()
                pltpu.async_copy(hr_h.at[sidx.at[ib]], rows_v.at[vb],
                                 sem_g.at[vb])
                pltpu.async_copy(invb_h.at[cidx.at[ib]], srow_v.at[vb],
                                 sem_c.at[vb])

            def wait_gathers(vb, ib, hr_h=hr_h):
                pltpu.make_async_copy(hr_h.at[sidx.at[ib]], rows_v.at[vb],
                                      sem_g.at[vb]).wait()
                pltpu.make_async_copy(invb_h.at[cidx.at[ib]], srow_v.at[vb],
                                      sem_c.at[vb]).wait()

            def wait_scatter(vb, ib):
                pltpu.make_async_copy(rows_v.at[vb],
                                      acc_sh.at[didx.at[ib]],
                                      sem_s.at[vb]).wait()

            pltpu.sync_copy(z_h.at[pl.ds(rb, _NPT)],
                            acc_sh.at[pl.ds(rb, _NPT)])
            plsc.subcore_barrier()
            for j in range(3):
                idx_dma(j, j)
            for j in range(2):
                start_gathers(j, j)

            def chunk(i, carry):
                vb = lax.bitwise_and(i, _NB - 1)
                ib = lax.bitwise_and(i, _NI - 1)

                @pl.when(i + 3 < _NCH)
                def _():
                    idx_dma(i + 3, lax.bitwise_and(i + 3, _NI - 1))

                @pl.when(i + 2 < _NCH)
                def _():
                    nvb = lax.bitwise_and(i + 2, _NB - 1)

                    @pl.when(i >= 2)
                    def _():
                        wait_scatter(nvb, lax.bitwise_and(i - 2, _NI - 1))
                    start_gathers(lax.bitwise_and(i + 2, _NI - 1), nvb)

                wait_gathers(vb, ib)

                def rowgrp(g, c2):
                    for jj in range(4):
                        row = g * 4 + jj
                        srow = srow_v[vb, row, :]
                        for k3 in range(_DH // 16):
                            sl = pl.ds(k3 * 16, 16)
                            rows_v[vb, row, sl] = rows_v[vb, row, sl] * srow
                    return c2

                lax.fori_loop(0, _C // 4, rowgrp, 0)
                pltpu.async_copy(rows_v.at[vb], acc_sh.at[didx.at[ib]],
                                 sem_s.at[vb], add=True)
                return carry

            lax.fori_loop(0, _NCH, chunk, 0)
            for j in range(_NB):
                i = _NCH - _NB + j
                wait_scatter(i & (_NB - 1), i & (_NI - 1))
            plsc.subcore_barrier()
            pltpu.sync_copy(acc_sh.at[pl.ds(rb, _NPT)],
                            out_h.at[half, c, pl.ds(rb, _NPT)])
            plsc.subcore_barrier()

    return k


def _agg(hra, hrb, invb, gsrc, gdst, dstv, zeros_ndh):
    return _agg_kernel()(hra, hrb, invb, gsrc, gdst, dstv, zeros_ndh)


def kernel(x, edge_index, edge_type, W1_rel, W1_root, b1,
           W2_rel, W2_root, b2, Wc, bc):
    src = edge_index[0]
    dst = edge_index[1]

    rows = _E // _D
    gsrc2, gdst2 = _gidx(src.reshape(rows, _D), dst.reshape(rows, _D),
                         edge_type.reshape(rows, _D))
    gsrc = gsrc2.reshape(_E)
    gdst = gdst2.reshape(_E)

    zeros_rnp = jnp.zeros((_RNP,), jnp.float32)
    zeros_ndh = jnp.zeros((_N, _DH), jnp.float32)

    cnt = _counts(gdst, zeros_rnp)
    inv = _inv(cnt.reshape(_NC, _RNP // _D, _D)).reshape(_RNP)
    invb = jnp.broadcast_to(inv[:, None], (_RNP, 16))

    wcat1 = jnp.concatenate([W1_rel, W1_root[None]], axis=0)
    bcat1 = jnp.zeros((_R + 1, 1, _D), jnp.float32).at[_R, 0].set(b1)
    ha1, hb1 = _mm(x, wcat1, bcat1)
    hra1 = ha1[:_R].reshape(_RN, _DH)
    hrb1 = hb1[:_R].reshape(_RN, _DH)
    agg1 = _agg(hra1, hrb1, invb, gsrc, gdst, dst, zeros_ndh)

    wcat2 = jnp.concatenate([W2_rel, W2_root[None]], axis=0)
    bcat2 = jnp.zeros((_R + 1, 1, _D), jnp.float32).at[_R, 0].set(b2)
    ha2, hb2 = _mid(ha1, hb1, agg1, wcat2, bcat2)
    hra2 = ha2[:_R].reshape(_RN, _DH)
    hrb2 = hb2[:_R].reshape(_RN, _DH)
    agg2 = _agg(hra2, hrb2, invb, gsrc, gdst, dst, zeros_ndh)

    wc_pad = jnp.pad(Wc, ((0, 0), (0, _D - Wc.shape[1])))
    bc_pad = jnp.pad(bc, (0, _D - bc.shape[0])).reshape(1, _D)
    out = _fin(ha2, hb2, agg2, wc_pad, bc_pad)
    return out[:, :Wc.shape[1]]
